# Initial kernel scaffold; baseline (speedup 1.0000x reference)
import jax
import jax.numpy as jnp
from jax.experimental import pallas as pl


def kernel(x, W1, W2, W3, W4, W5, W6, W7, W8, W9):
    raise NotImplementedError("write your pallas kernel here")


# speedup vs baseline: 7.9840x; 7.9840x over previous
import functools

import jax
import jax.numpy as jnp
from jax import lax
from jax.experimental import pallas as pl
from jax.experimental.pallas import tpu as pltpu
from jax.experimental.pallas import tpu_sc as plsc

B = 4
N = 2048
K = 20
EPS = 1e-5


def _lrelu(x):
    return jnp.where(x >= 0, x, 0.2 * x)


_RB = 512


def _knn_body(xr_ref, xc_ref, out_ref):
    b = pl.program_id(0)
    xb = xr_ref[0]
    xc = xc_ref[0]
    p = jnp.dot(xb, xc, preferred_element_type=jnp.float32)
    xxr = jnp.sum(xb * xb, axis=1, keepdims=True)
    xxc = jnp.sum(xc * xc, axis=0, keepdims=True)
    pw = 2.0 * p - xxr - xxc
    iota = lax.broadcasted_iota(jnp.int32, (_RB, N), 1)
    cols = []
    v = pw
    for _ in range(K):
        m = jnp.max(v, axis=1, keepdims=True)
        cand = jnp.where(v >= m, iota, N)
        jm = jnp.min(cand, axis=1, keepdims=True)
        cols.append(jm)
        v = jnp.where(iota == jm, -jnp.inf, v)
    idx = jnp.concatenate(cols, axis=1)
    out_ref[0] = idx + b * N


def _knn(xr, xc):
    return pl.pallas_call(
        _knn_body,
        grid=(B, N // _RB),
        in_specs=[
            pl.BlockSpec((1, _RB, 8), lambda b, r: (b, r, 0)),
            pl.BlockSpec((1, 8, N), lambda b, r: (b, 0, 0)),
        ],
        out_specs=pl.BlockSpec((1, _RB, K), lambda b, r: (b, r, 0)),
        out_shape=jax.ShapeDtypeStruct((B, N, K), jnp.int32),
    )(xr, xc)


_NC, _NS = 2, 16
_NW = _NC * _NS
_TOT = B * N * K
_PER_W = _TOT // _NW
_CH = 128
_NCHUNK = _PER_W // _CH


def _sc_gather_body(tbl_hbm, idx_hbm, out_hbm, idx_v, rows_v, sem):
    wid = lax.axis_index("s") * _NC + lax.axis_index("c")
    base = wid * _PER_W

    def step(i, carry):
        off = base + i * _CH
        pltpu.sync_copy(idx_hbm.at[pl.ds(off, _CH)], idx_v)
        pltpu.async_copy(tbl_hbm.at[idx_v], rows_v, sem).wait()
        pltpu.sync_copy(rows_v, out_hbm.at[pl.ds(off, _CH)])
        return carry

    lax.fori_loop(0, _NCHUNK, step, 0)


@functools.cache
def _sc_gather_call():
    return pl.kernel(
        _sc_gather_body,
        out_type=jax.ShapeDtypeStruct((_TOT, 128), jnp.float32),
        mesh=plsc.VectorSubcoreMesh(core_axis_name="c", subcore_axis_name="s",
                                    num_cores=_NC, num_subcores=_NS),
        scratch_types=[
            pltpu.VMEM((_CH,), jnp.int32),
            pltpu.VMEM((_CH, 128), jnp.float32),
            pltpu.SemaphoreType.DMA,
        ],
    )


def _sc_gather(table, flat_idx):
    return _sc_gather_call()(table, flat_idx)


def _edge_stats_norm(h):
    m = jnp.mean(h, axis=0, keepdims=True)
    v = jnp.mean((h - m) ** 2, axis=0, keepdims=True)
    return _lrelu((h - m) * lax.rsqrt(v + EPS))


def _max_over_k(h):
    m = h[0:N]
    for j in range(1, K):
        m = jnp.maximum(m, h[j * N:(j + 1) * N])
    return m


_RC = 2 * N
_NCH = (K * N) // _RC


def _edge_loop_body(g_hbm, z_ref, w1_ref, w2_ref, out_ref, s_ref, buf_ref,
                    sem, *, cin):
    b = pl.program_id(0)
    z = z_ref[0]
    zrep = jnp.concatenate([z, z], axis=0)
    scale = jnp.float32(1.0 / (K * N))

    ones = jnp.ones((1, N), jnp.float32)

    def _tree(vals):
        vals = vals + [jnp.zeros_like(vals[0])] * (32 - len(vals))
        while len(vals) > 1:
            h = len(vals) // 2
            vals = [vals[i] + vals[i + h] for i in range(h)]
        return vals[0]

    def stats_mean(readj):
        sj = [jnp.dot(ones, readj(j), precision=lax.Precision.HIGHEST,
                      preferred_element_type=jnp.float32) for j in range(K)]
        return _tree(sj) * scale

    def stats_var(readj, m):
        sj = []
        for j in range(K):
            d = readj(j) - m
            sj.append(jnp.dot(ones, d * d, precision=lax.Precision.HIGHEST,
                              preferred_element_type=jnp.float32))
        return _tree(sj) * scale

    acc = None
    for c in range(_NCH):
        cp = pltpu.make_async_copy(
            g_hbm.at[b, pl.ds(c * _RC, _RC), :], buf_ref, sem)
        cp.start()
        cp.wait()
        gc = buf_ref[:, :cin]
        f = jnp.concatenate([gc - zrep, zrep], axis=1)
        h = jnp.dot(f, w1_ref[...], preferred_element_type=jnp.float32)
        s_ref[pl.ds(c * _RC, _RC), :] = h
        if w2_ref is None:
            loc = jnp.maximum(h[0:N], h[N:2 * N])
            acc = loc if acc is None else jnp.maximum(acc, loc)

    read_s = lambda c: s_ref[pl.ds(c * _RC, _RC), :]
    read_j = lambda j: s_ref[pl.ds(j * N, N), :]
    m = stats_mean(read_j)
    sd = jnp.sqrt(stats_var(read_j, m) + EPS)

    if w2_ref is not None:
        for c in range(_NCH):
            h = _lrelu((read_s(c) - m) / sd)
            h = jnp.dot(h, w2_ref[...], preferred_element_type=jnp.float32)
            s_ref[pl.ds(c * _RC, _RC), :] = h
            loc = jnp.maximum(h[0:N], h[N:2 * N])
            acc = loc if acc is None else jnp.maximum(acc, loc)
        m = stats_mean(read_j)
        sd = jnp.sqrt(stats_var(read_j, m) + EPS)

    out_ref[0] = _lrelu((acc - m) / sd)


def _edge2_body(g_hbm, z_ref, w1_ref, w2_ref, out_ref, s_ref, buf_ref, sem,
                *, cin):
    _edge_loop_body(g_hbm, z_ref, w1_ref, w2_ref, out_ref, s_ref, buf_ref,
                    sem, cin=cin)


def _edge1_body(g_hbm, z_ref, w1_ref, out_ref, s_ref, buf_ref, sem, *, cin):
    _edge_loop_body(g_hbm, z_ref, w1_ref, None, out_ref, s_ref, buf_ref,
                    sem, cin=cin)


def _edge(g, z, w1_t, w2_t=None):
    cin = z.shape[2]
    in_specs = [
        pl.BlockSpec(memory_space=pl.ANY),
        pl.BlockSpec((1, N, cin), lambda b: (b, 0, 0)),
        pl.BlockSpec((2 * cin, 64), lambda b: (0, 0)),
    ]
    args = [g, z, w1_t]
    if w2_t is not None:
        in_specs.append(pl.BlockSpec((64, 64), lambda b: (0, 0)))
        args.append(w2_t)
        body = functools.partial(_edge2_body, cin=cin)
    else:
        body = functools.partial(_edge1_body, cin=cin)
    return pl.pallas_call(
        body,
        grid=(B,),
        in_specs=in_specs,
        out_specs=pl.BlockSpec((1, N, 64), lambda b: (b, 0, 0)),
        out_shape=jax.ShapeDtypeStruct((B, N, 64), jnp.float32),
        scratch_shapes=[
            pltpu.VMEM((K * N, 64), jnp.float32),
            pltpu.VMEM((_RC, 128), jnp.float32),
            pltpu.SemaphoreType.DMA,
        ],
    )(*args)


def _head_body(x1_ref, x2_ref, x3_ref, w6_ref, w7g_ref, w7x_ref, w8_ref,
               w9_ref, out_ref):
    cat = jnp.concatenate([x1_ref[0], x2_ref[0], x3_ref[0]], axis=1)
    h6 = jnp.dot(cat, w6_ref[...],
                 preferred_element_type=jnp.float32)
    h6 = _edge_stats_norm(h6)
    g = jnp.max(h6, axis=0, keepdims=True)
    gw = jnp.dot(g, w7g_ref[...],
                 preferred_element_type=jnp.float32)
    h7 = jnp.dot(cat, w7x_ref[...],
                 preferred_element_type=jnp.float32) + gw
    h7 = _edge_stats_norm(h7)
    h8 = jnp.dot(h7, w8_ref[...],
                 preferred_element_type=jnp.float32)
    h8 = _edge_stats_norm(h8)
    out_ref[0] = jnp.dot(h8, w9_ref[...],
                         preferred_element_type=jnp.float32)


def _head(x1, x2, x3, w6_t, w7g_t, w7x_t, w8_t, w9_t):
    return pl.pallas_call(
        _head_body,
        grid=(B,),
        in_specs=[
            pl.BlockSpec((1, N, 64), lambda b: (b, 0, 0)),
            pl.BlockSpec((1, N, 64), lambda b: (b, 0, 0)),
            pl.BlockSpec((1, N, 64), lambda b: (b, 0, 0)),
            pl.BlockSpec((192, 1024), lambda b: (0, 0)),
            pl.BlockSpec((1024, 512), lambda b: (0, 0)),
            pl.BlockSpec((192, 512), lambda b: (0, 0)),
            pl.BlockSpec((512, 256), lambda b: (0, 0)),
            pl.BlockSpec((256, 2), lambda b: (0, 0)),
        ],
        out_specs=pl.BlockSpec((1, N, 2), lambda b: (b, 0, 0)),
        out_shape=jax.ShapeDtypeStruct((B, N, 2), jnp.float32),
    )(x1, x2, x3, w6_t, w7g_t, w7x_t, w8_t, w9_t)


def _pad_lanes(a, c):
    return jnp.pad(a, ((0, 0), (0, 0), (0, c - a.shape[2])))


def _layer_gather(xr):
    xq = _pad_lanes(xr[:, :, :3], 8)
    xqc = jnp.transpose(xq, (0, 2, 1))
    idx = _knn(xq, xqc)
    flat_idx = jnp.transpose(idx, (0, 2, 1)).reshape(-1)
    tbl = _pad_lanes(xr, 128).reshape(B * N, 128)
    g = _sc_gather(tbl, flat_idx)
    return g.reshape(B, K * N, 128)


def kernel(x, W1, W2, W3, W4, W5, W6, W7, W8, W9):
    x = x.astype(jnp.float32)

    w6_t = W6.T
    w7g_t, w7x_t = W7[:, :1024].T, W7[:, 1024:].T

    x1 = _edge(_layer_gather(x), x, W1.T, W2.T)
    x2 = _edge(_layer_gather(x1), x1, W3.T, W4.T)
    x3 = _edge(_layer_gather(x2), x2, W5.T, None)

    return _head(x1, x2, x3, w6_t, w7g_t, w7x_t, W8.T, W9.T)
